# baseline (speedup 1.0000x reference)
import functools

import jax
import jax.numpy as jnp
from jax import lax
from jax.experimental import pallas as pl
from jax.experimental.pallas import tpu as pltpu
from jax.experimental.pallas import tpu_sc as plsc

_N = 4096
_C = 32
_NC = 2
_NS = 16
_NW = _NC * _NS
_BPW = _N // _NW
_CH = 8


def _table_body(z_ref, tab_ref):
    z = z_ref[0, :]
    c = lax.broadcasted_iota(jnp.int32, (_C, 1), 0)
    tab_ref[...] = (z[None, :] == c).astype(jnp.float32)


def _build_table(z2):
    return pl.pallas_call(
        _table_body,
        out_shape=jax.ShapeDtypeStruct((_C, _N), jnp.float32),
    )(z2)


def _sc_body(tab_hbm, z_hbm, out_hbm, idx_v, sem, tsem):
    sid = lax.axis_index("s")
    wid = sid * _NC + lax.axis_index("c")
    base = wid * _BPW
    pltpu.sync_copy(z_hbm.at[pl.ds(base, _BPW)], idx_v)
    lanes = lax.iota(jnp.int32, 16)
    for i in range(_BPW):
        if i % 16 == 0:
            vec = idx_v[pl.ds(i, 16)]
        zi = jnp.squeeze(lax.slice(vec, (i % 16,), (i % 16 + 1,)))
        pltpu.async_copy(
            tab_hbm.at[pl.ds(zi, 1)], out_hbm.at[pl.ds(base + i, 1)], sem)
    pltpu.make_async_copy(
        out_hbm.at[pl.ds(base, _BPW)], out_hbm.at[pl.ds(base, _BPW)], sem
    ).wait()


_sc_gather = functools.partial(
    pl.kernel,
    out_type=jax.ShapeDtypeStruct((_N, _N), jnp.float32),
    mesh=plsc.VectorSubcoreMesh(core_axis_name="c", subcore_axis_name="s"),
    scratch_types=[
        pltpu.VMEM((_BPW,), jnp.int32),
        pltpu.SemaphoreType.DMA,
        pltpu.SemaphoreType.DMA,
    ],
)(_sc_body)


def kernel(Z):
    z = Z.reshape(-1).astype(jnp.int32)
    tab = _build_table(z.reshape(1, _N))
    return _sc_gather(tab, z)


# speedup vs baseline: 14.0885x; 14.0885x over previous
import functools

import jax
import jax.numpy as jnp
from jax import lax
from jax.experimental import pallas as pl
from jax.experimental.pallas import tpu as pltpu
from jax.experimental.pallas import tpu_sc as plsc

_N = 4096
_C = 32
_NC = 2
_NS = 16
_NW = _NC * _NS
_BPW = _N // _NW
_G = 8
_VROWS = 6
_UNROLL = 16


def _table_body(z_ref, tab_ref):
    z = z_ref[0, :]
    c = lax.broadcasted_iota(jnp.int32, (_C, 1), 0)
    tab_ref[...] = (z[None, :] == c).astype(jnp.float32)


def _build_table(z2):
    return pl.pallas_call(
        _table_body,
        out_shape=jax.ShapeDtypeStruct((_C, _N), jnp.float32),
    )(z2)


def _sc_body(tab_hbm, z_hbm, out_hbm, idx_v, zall_v, tab_v, st0, st1,
             sem, ws0, ws1):
    sid = lax.axis_index("s")
    wid = sid * _NC + lax.axis_index("c")
    base = wid * _BPW
    pltpu.sync_copy(z_hbm.at[pl.ds(base, _BPW)], idx_v)
    pltpu.sync_copy(z_hbm, zall_v)
    @pl.when(sid == 0)
    def _():
        pltpu.sync_copy(tab_hbm, tab_v)
    plsc.subcore_barrier()
    sts = (st0, st1)
    wsems = (ws0, ws1)
    ones = jnp.full((16,), 1.0, jnp.float32)
    zeros = jnp.full((16,), 0.0, jnp.float32)

    def super_body(it, carry):
        vec = idx_v[pl.ds(it * 16, 16)]
        for k in range(2):
            st = sts[k]
            wsem = wsems[k]
            row0 = base + it * 16 + k * _G

            @pl.when(it > 0)
            def _():
                pltpu.make_async_copy(
                    out_hbm.at[pl.ds(base, _G)], st, wsem).wait()

            for j in range(_VROWS, _G):
                lane = k * _G + j
                zi = jnp.squeeze(lax.slice(vec, (lane,), (lane + 1,)))
                pltpu.async_copy(
                    tab_v.at[pl.ds(zi, 1)], st.at[pl.ds(j, 1)], sem)
            for j in range(_VROWS):
                lane = k * _G + j
                zi = jnp.squeeze(lax.slice(vec, (lane,), (lane + 1,)))
                zsplat = jnp.full((16,), zi, jnp.int32)

                def chunk_body(c, carry2, st=st, j=j, zsplat=zsplat):
                    for u in range(_UNROLL):
                        col = (c * _UNROLL + u) * 16
                        v = zall_v[pl.ds(col, 16)]
                        st[j, pl.ds(col, 16)] = jnp.where(
                            v == zsplat, ones, zeros)
                    return carry2

                lax.fori_loop(0, _N // 16 // _UNROLL, chunk_body, 0)
            if _VROWS < _G:
                pltpu.make_async_copy(
                    tab_v.at[pl.ds(0, _G - _VROWS)],
                    st.at[pl.ds(_VROWS, _G - _VROWS)], sem).wait()
            pltpu.async_copy(st, out_hbm.at[pl.ds(row0, _G)], wsem)
        return carry

    lax.fori_loop(0, _BPW // 16, super_body, 0)
    for k in range(2):
        pltpu.make_async_copy(
            out_hbm.at[pl.ds(base, _G)], sts[k], wsems[k]).wait()


_sc_gather = functools.partial(
    pl.kernel,
    out_type=jax.ShapeDtypeStruct((_N, _N), jnp.float32),
    mesh=plsc.VectorSubcoreMesh(core_axis_name="c", subcore_axis_name="s"),
    scratch_types=[
        pltpu.VMEM((_BPW,), jnp.int32),
        pltpu.VMEM((_N,), jnp.int32),
        pltpu.VMEM_SHARED((_C, _N), jnp.float32),
        pltpu.VMEM((_G, _N), jnp.float32),
        pltpu.VMEM((_G, _N), jnp.float32),
        pltpu.SemaphoreType.DMA,
        pltpu.SemaphoreType.DMA,
        pltpu.SemaphoreType.DMA,
    ],
)(_sc_body)


def kernel(Z):
    z = Z.reshape(-1).astype(jnp.int32)
    tab = _build_table(z.reshape(1, _N))
    return _sc_gather(tab, z)


# speedup vs baseline: 26.2620x; 1.8641x over previous
import functools

import jax
import jax.numpy as jnp
from jax import lax
from jax.experimental import pallas as pl
from jax.experimental.pallas import tpu as pltpu
from jax.experimental.pallas import tpu_sc as plsc

_N = 4096
_C = 32
_NC = 2
_NS = 16
_NW = _NC * _NS
_BPW = _N // _NW
_G = 8
_VROWS = 2
_UNROLL = 16


def _table_body(z_ref, tab_ref):
    z = z_ref[0, :]
    c = lax.broadcasted_iota(jnp.int32, (_C, 1), 0)
    tab_ref[...] = (z[None, :] == c).astype(jnp.float32)


def _build_table(z2):
    return pl.pallas_call(
        _table_body,
        out_shape=jax.ShapeDtypeStruct((_C, _N), jnp.float32),
    )(z2)


def _sc_body(tab_hbm, z_hbm, out_hbm, idx_v, zall_v, tab_v, st0, st1,
             sem, ws0, ws1):
    sid = lax.axis_index("s")
    wid = sid * _NC + lax.axis_index("c")
    base = wid * _BPW
    pltpu.sync_copy(z_hbm.at[pl.ds(base, _BPW)], idx_v)
    pltpu.sync_copy(z_hbm, zall_v)
    @pl.when(sid == 0)
    def _():
        pltpu.sync_copy(tab_hbm, tab_v)
    plsc.subcore_barrier()
    sts = (st0, st1)
    wsems = (ws0, ws1)
    ones = jnp.full((16,), 1.0, jnp.float32)
    zeros = jnp.full((16,), 0.0, jnp.float32)

    def super_body(it, carry):
        vec = idx_v[pl.ds(it * 16, 16)]
        for k in range(2):
            st = sts[k]
            wsem = wsems[k]
            row0 = base + it * 16 + k * _G

            @pl.when(it > 0)
            def _():
                pltpu.make_async_copy(
                    out_hbm.at[pl.ds(base, _G)], st, wsem).wait()

            for j in range(_VROWS, _G):
                lane = k * _G + j
                zi = jnp.squeeze(lax.slice(vec, (lane,), (lane + 1,)))
                pltpu.async_copy(
                    tab_v.at[pl.ds(zi, 1)], st.at[pl.ds(j, 1)], sem)
            for j in range(_VROWS):
                lane = k * _G + j
                zi = jnp.squeeze(lax.slice(vec, (lane,), (lane + 1,)))
                zsplat = jnp.full((16,), zi, jnp.int32)

                def chunk_body(c, carry2, st=st, j=j, zsplat=zsplat):
                    for u in range(_UNROLL):
                        col = (c * _UNROLL + u) * 16
                        v = zall_v[pl.ds(col, 16)]
                        st[j, pl.ds(col, 16)] = jnp.where(
                            v == zsplat, ones, zeros)
                    return carry2

                lax.fori_loop(0, _N // 16 // _UNROLL, chunk_body, 0)
            if _VROWS < _G:
                pltpu.make_async_copy(
                    tab_v.at[pl.ds(0, _G - _VROWS)],
                    st.at[pl.ds(_VROWS, _G - _VROWS)], sem).wait()
            pltpu.async_copy(st, out_hbm.at[pl.ds(row0, _G)], wsem)
        return carry

    lax.fori_loop(0, _BPW // 16, super_body, 0)
    for k in range(2):
        pltpu.make_async_copy(
            out_hbm.at[pl.ds(base, _G)], sts[k], wsems[k]).wait()


_sc_gather = functools.partial(
    pl.kernel,
    out_type=jax.ShapeDtypeStruct((_N, _N), jnp.float32),
    mesh=plsc.VectorSubcoreMesh(core_axis_name="c", subcore_axis_name="s"),
    scratch_types=[
        pltpu.VMEM((_BPW,), jnp.int32),
        pltpu.VMEM((_N,), jnp.int32),
        pltpu.VMEM_SHARED((_C, _N), jnp.float32),
        pltpu.VMEM((_G, _N), jnp.float32),
        pltpu.VMEM((_G, _N), jnp.float32),
        pltpu.SemaphoreType.DMA,
        pltpu.SemaphoreType.DMA,
        pltpu.SemaphoreType.DMA,
    ],
)(_sc_body)


def kernel(Z):
    z = Z.reshape(-1).astype(jnp.int32)
    tab = _build_table(z.reshape(1, _N))
    return _sc_gather(tab, z)


# speedup vs baseline: 37.3615x; 1.4226x over previous
import functools

import jax
import jax.numpy as jnp
from jax import lax
from jax.experimental import pallas as pl
from jax.experimental.pallas import tpu as pltpu
from jax.experimental.pallas import tpu_sc as plsc

_N = 4096
_C = 32
_NC = 2
_NS = 16
_NW = _NC * _NS
_BPW = _N // _NW
_G = 8


def _table_body(z_ref, tab_ref):
    z = z_ref[0, :]
    c = lax.broadcasted_iota(jnp.int32, (_C, 1), 0)
    tab_ref[...] = (z[None, :] == c).astype(jnp.float32)


def _build_table(z2):
    return pl.pallas_call(
        _table_body,
        out_shape=jax.ShapeDtypeStruct((_C, _N), jnp.float32),
    )(z2)


def _sc_body(tab_hbm, z_hbm, out_hbm, idx_v, tab_v, st0, st1, sem, ws0, ws1):
    sid = lax.axis_index("s")
    wid = sid * _NC + lax.axis_index("c")
    base = wid * _BPW
    pltpu.sync_copy(z_hbm.at[pl.ds(base, _BPW)], idx_v)
    @pl.when(sid == 0)
    def _():
        pltpu.sync_copy(tab_hbm, tab_v)
    plsc.subcore_barrier()
    sts = (st0, st1)
    wsems = (ws0, ws1)
    n_groups = _BPW // _G
    wcp = [None] * n_groups
    for g in range(n_groups):
        st = sts[g % 2]
        if g >= 2:
            wcp[g - 2].wait()
        for j in range(_G):
            i = g * _G + j
            if i % 16 == 0:
                vec = idx_v[pl.ds(i, 16)]
            zi = jnp.squeeze(lax.slice(vec, (i % 16,), (i % 16 + 1,)))
            pltpu.async_copy(
                tab_v.at[pl.ds(zi, 1)], st.at[pl.ds(j, 1)], sem)
        pltpu.make_async_copy(tab_v.at[pl.ds(0, _G)], st, sem).wait()
        wcp[g] = pltpu.async_copy(
            st, out_hbm.at[pl.ds(base + g * _G, _G)], wsems[g % 2])
    wcp[n_groups - 2].wait()
    wcp[n_groups - 1].wait()


_sc_gather = functools.partial(
    pl.kernel,
    out_type=jax.ShapeDtypeStruct((_N, _N), jnp.float32),
    mesh=plsc.VectorSubcoreMesh(core_axis_name="c", subcore_axis_name="s"),
    scratch_types=[
        pltpu.VMEM((_BPW,), jnp.int32),
        pltpu.VMEM_SHARED((_C, _N), jnp.float32),
        pltpu.VMEM((_G, _N), jnp.float32),
        pltpu.VMEM((_G, _N), jnp.float32),
        pltpu.SemaphoreType.DMA,
        pltpu.SemaphoreType.DMA,
        pltpu.SemaphoreType.DMA,
    ],
)(_sc_body)


def kernel(Z):
    z = Z.reshape(-1).astype(jnp.int32)
    tab = _build_table(z.reshape(1, _N))
    return _sc_gather(tab, z)
